# baseline (speedup 1.0000x reference)
import jax
import jax.numpy as jnp
from jax.experimental import pallas as pl
from jax.experimental.pallas import tpu as pltpu

_BETA = 0.25
_B = 8192
_D = 1024
_K = 1024
_BLK = 256
_KT = 256
_DT = 256


def _rkm_block(
    x_ref,
    cb0_ref,
    cb1_ref,
    cb2_ref,
    nsq_ref,
    out_ref,
    idx_ref,
    loss_ref,
    r_s,
    d_s,
):
    x = x_ref[...]
    r_s[...] = x
    idxs = []
    loss_row = jnp.zeros((_BLK,), jnp.float32)
    for lvl, cb_ref in enumerate((cb0_ref, cb1_ref, cb2_ref)):
        r = r_s[...]
        rsq = jnp.sum(r * r, axis=1, keepdims=True)
        for kb in range(_K // _KT):
            xc_t = jax.lax.dot_general(
                r,
                cb_ref[kb * _KT : (kb + 1) * _KT, :],
                (((1,), (1,)), ((), ())),
                preferred_element_type=jnp.float32,
                precision=jax.lax.Precision.DEFAULT,
            )
            d_s[:, kb * _KT : (kb + 1) * _KT] = (
                rsq - 2.0 * xc_t
            ) + nsq_ref[lvl, kb * _KT : (kb + 1) * _KT]
        d = d_s[...]
        m = jnp.min(d, axis=1)
        jix = jax.lax.broadcasted_iota(jnp.int32, (_BLK, _K), 1)
        idx = jnp.min(
            jnp.where(d == m[:, None], jix, jnp.int32(_K)), axis=1
        ).astype(jnp.int32)
        loss_row = loss_row + m
        onehot = (
            idx[:, None] == jax.lax.broadcasted_iota(jnp.int32, (1, _K), 1)
        ).astype(jnp.float32)
        for db in range(_D // _DT):
            sl = slice(db * _DT, (db + 1) * _DT)
            xq_t = jax.lax.dot_general(
                onehot,
                cb_ref[:, sl],
                (((1,), (0,)), ((), ())),
                preferred_element_type=jnp.float32,
                precision=jax.lax.Precision.HIGHEST,
            )
            rt = r_s[:, sl]
            xqst = rt + (xq_t - rt)
            if lvl == 0:
                out_ref[:, sl] = xqst
            else:
                out_ref[:, sl] += xqst
            r_s[:, sl] = rt - xqst
        idxs.append(idx)
    idx_ref[...] = jnp.stack(idxs + [idxs[0]] * 5, axis=0)
    loss_ref[...] = jnp.full((1, 1, 128), jnp.sum(loss_row), jnp.float32)


def kernel(x, cb0, cb1, cb2):
    nsq = jnp.stack(
        [
            jnp.sum(cb0**2, axis=1),
            jnp.sum(cb1**2, axis=1),
            jnp.sum(cb2**2, axis=1),
        ],
        axis=0,
    )
    nsq = jnp.concatenate([nsq, jnp.zeros((5, _K), jnp.float32)], axis=0)
    grid = (_B // _BLK,)
    n_steps = _B // _BLK
    out, idxp, loss = pl.pallas_call(
        _rkm_block,
        grid=grid,
        in_specs=[
            pl.BlockSpec((_BLK, _D), lambda i: (i, 0)),
            pl.BlockSpec((_K, _D), lambda i: (0, 0)),
            pl.BlockSpec((_K, _D), lambda i: (0, 0)),
            pl.BlockSpec((_K, _D), lambda i: (0, 0)),
            pl.BlockSpec((8, _K), lambda i: (0, 0)),
        ],
        out_specs=[
            pl.BlockSpec((_BLK, _D), lambda i: (i, 0)),
            pl.BlockSpec((8, _BLK), lambda i: (0, i)),
            pl.BlockSpec((1, 1, 128), lambda i: (i, 0, 0)),
        ],
        out_shape=[
            jax.ShapeDtypeStruct((_B, _D), jnp.float32),
            jax.ShapeDtypeStruct((8, _B), jnp.int32),
            jax.ShapeDtypeStruct((n_steps, 1, 128), jnp.float32),
        ],
        scratch_shapes=[
            pltpu.VMEM((_BLK, _D), jnp.float32),
            pltpu.VMEM((_BLK, _K), jnp.float32),
        ],
        compiler_params=pltpu.CompilerParams(
            dimension_semantics=("parallel",),
        ),
    )(x, cb0, cb1, cb2, nsq)
    rq_loss = jnp.sum(loss[:, 0, 0]) * ((1.0 + _BETA) / (3.0 * _B * _D))
    indices = idxp[:3, :].T
    return out, rq_loss, indices


# speedup vs baseline: 1.5026x; 1.5026x over previous
import jax
import jax.numpy as jnp
from jax.experimental import pallas as pl
from jax.experimental.pallas import tpu as pltpu

_BETA = 0.25
_B = 8192
_D = 1024
_K = 1024
_BLK = 256
_KT = 256
_DT = 256


def _trunc16(v):
    u = jax.lax.bitcast_convert_type(v, jnp.uint32)
    return jax.lax.bitcast_convert_type(
        u & jnp.uint32(0xFFFF0000), jnp.float32
    )


def _dot1p(a, b):
    return jax.lax.dot_general(
        a,
        b,
        (((1,), (0,)), ((), ())),
        preferred_element_type=jnp.float32,
        precision=jax.lax.Precision.DEFAULT,
    )


def _rkm_block(
    x_ref,
    cb0_ref,
    cb1_ref,
    cb2_ref,
    nsq_ref,
    out_ref,
    idx_ref,
    loss_ref,
    r_s,
    d_s,
):
    x = x_ref[...]
    r_s[...] = x
    idxs = []
    loss_row = jnp.zeros((_BLK,), jnp.float32)
    for lvl, cb_ref in enumerate((cb0_ref, cb1_ref, cb2_ref)):
        r = r_s[...]
        rsq = jnp.sum(r * r, axis=1, keepdims=True)
        for kb in range(_K // _KT):
            xc_t = jax.lax.dot_general(
                r,
                cb_ref[kb * _KT : (kb + 1) * _KT, :],
                (((1,), (1,)), ((), ())),
                preferred_element_type=jnp.float32,
                precision=jax.lax.Precision.DEFAULT,
            )
            d_s[:, kb * _KT : (kb + 1) * _KT] = (
                rsq - 2.0 * xc_t
            ) + nsq_ref[lvl, kb * _KT : (kb + 1) * _KT]
        d = d_s[...]
        m = jnp.min(d, axis=1)
        jix = jax.lax.broadcasted_iota(jnp.int32, (_BLK, _K), 1)
        idx = jnp.min(
            jnp.where(d == m[:, None], jix, jnp.int32(_K)), axis=1
        ).astype(jnp.int32)
        loss_row = loss_row + m
        onehot = (
            idx[:, None] == jax.lax.broadcasted_iota(jnp.int32, (1, _K), 1)
        ).astype(jnp.float32)
        for db in range(_D // _DT):
            sl = slice(db * _DT, (db + 1) * _DT)
            cbt = cb_ref[:, sl]
            hi = _trunc16(cbt)
            d1 = cbt - hi
            mid = _trunc16(d1)
            lo = d1 - mid
            xq_t = (
                _dot1p(onehot, hi) + _dot1p(onehot, mid)
            ) + _dot1p(onehot, lo)
            rt = r_s[:, sl]
            xqst = rt + (xq_t - rt)
            if lvl == 0:
                out_ref[:, sl] = xqst
            else:
                out_ref[:, sl] += xqst
            r_s[:, sl] = rt - xqst
        idxs.append(idx)
    idx_ref[...] = jnp.stack(idxs + [idxs[0]] * 5, axis=0)
    loss_ref[...] = jnp.full((1, 1, 128), jnp.sum(loss_row), jnp.float32)


def kernel(x, cb0, cb1, cb2):
    nsq = jnp.stack(
        [
            jnp.sum(cb0**2, axis=1),
            jnp.sum(cb1**2, axis=1),
            jnp.sum(cb2**2, axis=1),
        ],
        axis=0,
    )
    nsq = jnp.concatenate([nsq, jnp.zeros((5, _K), jnp.float32)], axis=0)
    grid = (_B // _BLK,)
    n_steps = _B // _BLK
    out, idxp, loss = pl.pallas_call(
        _rkm_block,
        grid=grid,
        in_specs=[
            pl.BlockSpec((_BLK, _D), lambda i: (i, 0)),
            pl.BlockSpec((_K, _D), lambda i: (0, 0)),
            pl.BlockSpec((_K, _D), lambda i: (0, 0)),
            pl.BlockSpec((_K, _D), lambda i: (0, 0)),
            pl.BlockSpec((8, _K), lambda i: (0, 0)),
        ],
        out_specs=[
            pl.BlockSpec((_BLK, _D), lambda i: (i, 0)),
            pl.BlockSpec((8, _BLK), lambda i: (0, i)),
            pl.BlockSpec((1, 1, 128), lambda i: (i, 0, 0)),
        ],
        out_shape=[
            jax.ShapeDtypeStruct((_B, _D), jnp.float32),
            jax.ShapeDtypeStruct((8, _B), jnp.int32),
            jax.ShapeDtypeStruct((n_steps, 1, 128), jnp.float32),
        ],
        scratch_shapes=[
            pltpu.VMEM((_BLK, _D), jnp.float32),
            pltpu.VMEM((_BLK, _K), jnp.float32),
        ],
        compiler_params=pltpu.CompilerParams(
            dimension_semantics=("parallel",),
        ),
    )(x, cb0, cb1, cb2, nsq)
    rq_loss = jnp.sum(loss[:, 0, 0]) * ((1.0 + _BETA) / (3.0 * _B * _D))
    indices = idxp[:3, :].T
    return out, rq_loss, indices
